# Initial kernel scaffold; baseline (speedup 1.0000x reference)
import jax
import jax.numpy as jnp
from jax.experimental import pallas as pl


def kernel(mu, pw_samples):
    raise NotImplementedError("write your pallas kernel here")


# speedup vs baseline: 1.0774x; 1.0774x over previous
import functools

import jax
import jax.numpy as jnp
import numpy as np
from jax import lax
from jax.experimental import pallas as pl
from jax.experimental.pallas import tpu as pltpu
from jax.experimental.pallas import tpu_sc as plsc

_LO = np.float32(-0.99999994)
_SPAN = np.float32(np.float32(1.0) - _LO)
_SQRT2 = np.float32(np.sqrt(2.0))


def _threefry2x32(k0, k1, x0, x1):
    ks2 = k0 ^ k1 ^ jnp.uint32(0x1BD11BDA)
    ks = (k0, k1, ks2)
    rots = ((13, 15, 26, 6), (17, 29, 16, 24))

    def rotl(x, d):
        return lax.shift_left(x, jnp.uint32(d)) | lax.shift_right_logical(
            x, jnp.uint32(32 - d))

    x0 = x0 + k0
    x1 = x1 + k1
    for i in range(5):
        for r in rots[i % 2]:
            x0 = x0 + x1
            x1 = rotl(x1, r)
            x1 = x1 ^ x0
        x0 = x0 + ks[(i + 1) % 3]
        x1 = x1 + ks[(i + 2) % 3] + jnp.uint32(i + 1)
    return x0, x1


def _erfinv_f32(x):
    w = -jnp.log1p(-x * x)
    w1 = w - jnp.float32(2.5)
    p = jnp.float32(2.81022636e-08)
    for c in (3.43273939e-07, -3.5233877e-06, -4.39150654e-06, 0.00021858087,
              -0.00125372503, -0.00417768164, 0.246640727, 1.50140941):
        p = jnp.float32(c) + p * w1
    w2 = jnp.sqrt(w) - jnp.float32(3.0)
    q = jnp.float32(-0.000200214257)
    for c in (0.000100950558, 0.00134934322, -0.00367342844, 0.00573950773,
              -0.0076224613, 0.00943887047, 1.00167406, 2.83297682):
        q = jnp.float32(c) + q * w2
    return jnp.where(w < jnp.float32(5.0), p, q) * x


def _bits_to_unit(bits):
    f = lax.bitcast_convert_type(
        lax.shift_right_logical(bits, jnp.uint32(9)) | jnp.uint32(0x3F800000),
        jnp.float32)
    return f - jnp.float32(1.0)


def _vmf_body(key_ref, w_ref, mu_ref, o_ref):
    R, D = mu_ref.shape
    i = pl.program_id(0)
    base = (i * (R * D)).astype(jnp.uint32)
    rows = lax.broadcasted_iota(jnp.int32, (R, D), 0).astype(jnp.uint32)
    cols = lax.broadcasted_iota(jnp.int32, (R, D), 1).astype(jnp.uint32)
    p = base + rows * jnp.uint32(D) + cols
    y0, y1 = _threefry2x32(key_ref[0], key_ref[1], jnp.zeros_like(p), p)
    u = jnp.maximum(_LO, _bits_to_unit(y0 ^ y1) * _SPAN + _LO)
    eps = _SQRT2 * _erfinv_f32(u)

    mu = mu_ref[...]
    d = jnp.sum(eps * mu, axis=1, keepdims=True)
    nu = eps - d * mu
    nn = jnp.maximum(jnp.sqrt(jnp.sum(nu * nu, axis=1, keepdims=True)),
                     jnp.float32(1e-12))
    w = w_ref[...]
    o_ref[...] = w * mu + jnp.sqrt(jnp.float32(1.0) - w * w) * (nu / nn)


@functools.lru_cache(maxsize=None)
def _build_tc_vmf(B, D, R):
    return pl.pallas_call(
        _vmf_body,
        grid=(B // R,),
        in_specs=[
            pl.BlockSpec(memory_space=pltpu.SMEM),
            pl.BlockSpec((R, 1), lambda i: (i, 0)),
            pl.BlockSpec((R, D), lambda i: (i, 0)),
        ],
        out_specs=pl.BlockSpec((R, D), lambda i: (i, 0)),
        out_shape=jax.ShapeDtypeStruct((B, D), jnp.float32),
    )


@functools.lru_cache(maxsize=None)
def _build_sc_gather(N, B):
    try:
        info = plsc.get_sparse_core_info()
        NC, NS = info.num_cores, info.num_subcores
    except Exception:
        NC, NS = 2, 16
    NW = NC * NS
    BW = B // NW
    mesh = plsc.VectorSubcoreMesh(core_axis_name="c", subcore_axis_name="s")

    @functools.partial(
        pl.kernel,
        out_type=jax.ShapeDtypeStruct((B,), jnp.float32),
        mesh=mesh,
        scratch_types=[
            pltpu.VMEM((BW,), jnp.int32),
            pltpu.VMEM((BW,), jnp.float32),
            pltpu.SemaphoreType.DMA,
        ],
    )
    def _gather(tab_hbm, idx_hbm, out_hbm, idx_v, w_v, sem):
        wid = lax.axis_index("s") * NC + lax.axis_index("c")
        base = wid * BW
        pltpu.sync_copy(idx_hbm.at[pl.ds(base, BW)], idx_v)
        pltpu.async_copy(tab_hbm.at[idx_v], w_v, sem).wait()
        pltpu.sync_copy(w_v, out_hbm.at[pl.ds(base, BW)])

    return _gather


def kernel(mu, pw_samples):
    B, D = mu.shape
    N = pw_samples.shape[0]
    k_idx = jax.random.fold_in(jax.random.key(1), 0)
    k_eps = jax.random.fold_in(jax.random.key(1), 1)
    idxs = jax.random.uniform(k_idx, (B, 1), minval=0.0,
                              maxval=float(N)).astype(jnp.int32)
    key_data = jax.random.key_data(k_eps).astype(jnp.uint32)

    w = _build_sc_gather(N, B)(pw_samples.reshape(N), idxs.reshape(B))
    return _build_tc_vmf(B, D, 256)(key_data, w.reshape(B, 1), mu)
